# baseline (speedup 1.0000x reference)
import functools

import jax
import jax.numpy as jnp
from jax import lax
from jax.experimental import pallas as pl
from jax.experimental.pallas import tpu as pltpu
from jax.experimental.pallas import tpu_sc as plsc

B, C, H, W, K = 32, 64, 128, 128, 128
HW = H * W
NC, NS, L = 2, 16, 16
NW = NC * NS
EPW = K * C
KB = K // L
BLK = L * C
GCH = 128
WEIGHT_ANGLE = 10.0

_mesh = plsc.VectorSubcoreMesh(core_axis_name="c", subcore_axis_name="s")


@functools.partial(
    pl.kernel,
    mesh=_mesh,
    out_type=(
        jax.ShapeDtypeStruct((NW, L), jnp.float32),
        jax.ShapeDtypeStruct((NW, L), jnp.float32),
    ),
    scratch_types=[
        pltpu.VMEM((K,), jnp.int32),
        pltpu.VMEM((K,), jnp.int32),
        pltpu.VMEM((EPW,), jnp.int32),
        pltpu.VMEM((EPW,), jnp.float32),
        pltpu.VMEM((EPW,), jnp.float32),
        pltpu.VMEM((L,), jnp.float32),
        pltpu.VMEM((L,), jnp.float32),
        pltpu.SemaphoreType.DMA,
        pltpu.SemaphoreType.DMA,
        pltpu.SemaphoreType.DMA,
    ],
)
def _sc_partials(out_hbm, ind_hbm, mask_hbm, tgt_hbm,
                 psum_hbm, pcnt_hbm,
                 ind_v, mask_v, idx_v, pred_v, tgt_v,
                 psum_v, pcnt_v, sem_t, sem_a, sem_b):
    wid = lax.axis_index("s") * NC + lax.axis_index("c")

    cp_t = pltpu.async_copy(tgt_hbm.at[pl.ds(wid * EPW, EPW)], tgt_v, sem_t)
    pltpu.sync_copy(ind_hbm.at[pl.ds(wid * K, K)], ind_v)
    pltpu.sync_copy(mask_hbm.at[pl.ds(wid * K, K)], mask_v)

    lanes = lax.iota(jnp.int32, L)
    base = wid * (C * HW)
    wvec = jnp.where(lanes % 2 == 0,
                     jnp.full((L,), 1.0, jnp.float32),
                     jnp.full((L,), WEIGHT_ANGLE, jnp.float32))
    sems = (sem_a, sem_b)

    def build(kb):
        bvec = ind_v[pl.ds(kb * L, L)] + base

        def bb(cc2, x):
            v0 = bvec + cc2 * (4 * HW)
            o0 = kb * BLK + cc2 * (4 * L)
            for u in range(4):
                idx_v[pl.ds(o0 + u * L, L)] = v0 + u * HW
            return x

        lax.fori_loop(0, C // 4, bb, 0)
        for ch in range(BLK // GCH):
            off = kb * BLK + ch * GCH
            pltpu.async_copy(out_hbm.at[idx_v.at[pl.ds(off, GCH)]],
                             pred_v.at[pl.ds(off, GCH)], sems[kb % 2])

    def drain(kb):
        pltpu.make_async_copy(out_hbm.at[pl.ds(0, BLK)],
                              pred_v.at[pl.ds(0, BLK)], sems[kb % 2]).wait()

    def compute(kb, acc, cnt):
        mf = mask_v[pl.ds(kb * L, L)].astype(jnp.float32)
        coef16 = mf * wvec

        def cb(cc2, a):
            o0 = kb * BLK + cc2 * (4 * L)
            for u in range(4):
                tg = tgt_v[pl.ds(o0 + u * L, L)]
                pr = pred_v[pl.ds(o0 + u * L, L)]
                a = a + coef16 * jnp.abs(pr - tg)
            return a

        acc = lax.fori_loop(0, C // 4, cb, acc)
        return acc, cnt + mf

    acc = jnp.zeros((L,), jnp.float32)
    cnt = jnp.zeros((L,), jnp.float32)
    build(0)
    cp_t.wait()
    for kb in range(1, KB):
        build(kb)
        drain(kb - 1)
        acc, cnt = compute(kb - 1, acc, cnt)
    drain(KB - 1)
    acc, cnt = compute(KB - 1, acc, cnt)

    psum_v[...] = acc
    pcnt_v[...] = cnt
    pltpu.sync_copy(psum_v, psum_hbm.at[wid])
    pltpu.sync_copy(pcnt_v, pcnt_hbm.at[wid])


def _finish_body(ps_ref, pc_ref, o_ref):
    total = jnp.sum(ps_ref[...])
    count = jnp.sum(pc_ref[...])
    o_ref[...] = jnp.broadcast_to(total / (count * float(C) + 1e-4), (1, 1))


_finish = pl.pallas_call(
    _finish_body,
    out_shape=jax.ShapeDtypeStruct((1, 1), jnp.float32),
)


def kernel(output, mask, ind, target, freq_mask):
    del freq_mask
    psum, pcnt = _sc_partials(
        output.reshape(-1),
        ind.reshape(-1).astype(jnp.int32),
        mask.reshape(-1).astype(jnp.int32),
        target.reshape(B, KB, L, C).transpose(0, 1, 3, 2).reshape(-1),
    )
    return _finish(psum, pcnt)[0, 0]


# speedup vs baseline: 1.1880x; 1.1880x over previous
import functools

import jax
import jax.numpy as jnp
from jax import lax
from jax.experimental import pallas as pl
from jax.experimental.pallas import tpu as pltpu
from jax.experimental.pallas import tpu_sc as plsc

B, C, H, W, K = 32, 64, 128, 128, 128
HW = H * W
NC, NS, L = 2, 16, 16
NW = NC * NS
EPW = K * C
KB = K // L
BLK = L * C
GCH = 128
WEIGHT_ANGLE = 10.0

_mesh = plsc.VectorSubcoreMesh(core_axis_name="c", subcore_axis_name="s")


@functools.partial(
    pl.kernel,
    mesh=_mesh,
    out_type=(
        jax.ShapeDtypeStruct((NW, L), jnp.float32),
        jax.ShapeDtypeStruct((NW, L), jnp.float32),
    ),
    scratch_types=[
        pltpu.VMEM((K,), jnp.int32),
        pltpu.VMEM((K,), jnp.int32),
        pltpu.VMEM((EPW,), jnp.int32),
        pltpu.VMEM((EPW,), jnp.float32),
        pltpu.VMEM((EPW,), jnp.float32),
        pltpu.VMEM((L,), jnp.float32),
        pltpu.VMEM((L,), jnp.float32),
        pltpu.SemaphoreType.DMA,
        pltpu.SemaphoreType.DMA,
        pltpu.SemaphoreType.DMA,
    ],
)
def _sc_partials(out_hbm, ind_hbm, mask_hbm, tgt_hbm,
                 psum_hbm, pcnt_hbm,
                 ind_v, mask_v, idx_v, pred_v, tgt_v,
                 psum_v, pcnt_v, sem_t, sem_a, sem_b):
    wid = lax.axis_index("s") * NC + lax.axis_index("c")

    cp_t = pltpu.async_copy(tgt_hbm.at[pl.ds(wid * EPW, EPW)], tgt_v, sem_t)
    pltpu.sync_copy(ind_hbm.at[pl.ds(wid * K, K)], ind_v)
    pltpu.sync_copy(mask_hbm.at[pl.ds(wid * K, K)], mask_v)

    lanes = lax.iota(jnp.int32, L)
    base = wid * (C * HW)
    wvec = jnp.where(lanes % 2 == 0,
                     jnp.full((L,), 1.0, jnp.float32),
                     jnp.full((L,), WEIGHT_ANGLE, jnp.float32))
    lhw = [(lanes + cb * L) * HW for cb in range(C // L)]
    sems = (sem_a, sem_b)

    def build(kb):
        vk = ind_v[pl.ds(kb * L, L)] + base
        for u in range(L):
            sk = vk[u]
            for cb in range(C // L):
                idx_v[pl.ds(kb * BLK + u * C + cb * L, L)] = lhw[cb] + sk
        for ch in range(BLK // GCH):
            off = kb * BLK + ch * GCH
            pltpu.async_copy(out_hbm.at[idx_v.at[pl.ds(off, GCH)]],
                             pred_v.at[pl.ds(off, GCH)], sems[kb % 2])

    def drain(kb):
        pltpu.make_async_copy(out_hbm.at[pl.ds(0, BLK)],
                              pred_v.at[pl.ds(0, BLK)], sems[kb % 2]).wait()

    def compute(kb, acc, cnt):
        mf = mask_v[pl.ds(kb * L, L)].astype(jnp.float32)
        coefv = mf * wvec
        for u in range(L):
            cf = jnp.full((L,), coefv[u], jnp.float32)
            for cb in range(C // L):
                off = kb * BLK + u * C + cb * L
                d = pred_v[pl.ds(off, L)] - tgt_v[pl.ds(off, L)]
                acc = acc + cf * jnp.abs(d)
        return acc, cnt + mf

    acc = jnp.zeros((L,), jnp.float32)
    cnt = jnp.zeros((L,), jnp.float32)
    build(0)
    cp_t.wait()
    for kb in range(1, KB):
        build(kb)
        drain(kb - 1)
        acc, cnt = compute(kb - 1, acc, cnt)
    drain(KB - 1)
    acc, cnt = compute(KB - 1, acc, cnt)

    psum_v[...] = acc
    pcnt_v[...] = cnt
    pltpu.sync_copy(psum_v, psum_hbm.at[wid])
    pltpu.sync_copy(pcnt_v, pcnt_hbm.at[wid])


def _finish_body(ps_ref, pc_ref, o_ref):
    total = jnp.sum(ps_ref[...])
    count = jnp.sum(pc_ref[...])
    o_ref[...] = jnp.broadcast_to(total / (count * float(C) + 1e-4), (1, 1))


_finish = pl.pallas_call(
    _finish_body,
    out_shape=jax.ShapeDtypeStruct((1, 1), jnp.float32),
)


def kernel(output, mask, ind, target, freq_mask):
    del freq_mask
    psum, pcnt = _sc_partials(
        output.reshape(-1),
        ind.reshape(-1).astype(jnp.int32),
        mask.reshape(-1).astype(jnp.int32),
        target.reshape(-1),
    )
    return _finish(psum, pcnt)[0, 0]
